# baseline (speedup 1.0000x reference)
import jax
import jax.numpy as jnp
from jax import lax
from jax.experimental import pallas as pl
from jax.experimental.pallas import tpu as pltpu
from jax.experimental.pallas import tpu_sc as plsc

B, N, M = 8, 2048, 2048
BR = 128
L = 16
TR = 16
R = B * N
NT = R // TR
NW = 32
TPW = NT // NW
TILES_PER_BATCH = N // TR
VPR = M // L


def _sc_zero_body(nrow_hbm, out_hbm, nrow_v, zbuf):
    wid = lax.axis_index("s") * 2 + lax.axis_index("c")

    pltpu.sync_copy(nrow_hbm, nrow_v)
    nrowf = nrow_v[...].astype(jnp.float32)
    lanes = jnp.arange(L, dtype=jnp.int32)

    z = jnp.zeros((L,), jnp.float32)

    def zrow(r, c):
        for k in range(VPR):
            zbuf[r, pl.ds(k * L, L)] = z
        return c

    lax.fori_loop(0, TR, zrow, jnp.int32(0))

    def tile_step(i, c):
        t = wid + NW * i
        b = t // TILES_PER_BATCH
        start = (t - b * TILES_PER_BATCH) * TR
        nrow_b = jnp.sum(jnp.where(lanes == b, nrowf, 0.0)).astype(jnp.int32)

        @pl.when(start >= nrow_b)
        def _():
            pltpu.sync_copy(zbuf, out_hbm.at[pl.ds(t * TR, TR)])

        return c

    lax.fori_loop(0, TPW, tile_step, jnp.int32(0))


def _sc_zero_fill(nrow16):
    mesh = plsc.VectorSubcoreMesh(core_axis_name="c", subcore_axis_name="s")
    return pl.kernel(
        _sc_zero_body,
        mesh=mesh,
        compiler_params=pltpu.CompilerParams(needs_layout_passes=False),
        out_type=jax.ShapeDtypeStruct((R, M), jnp.float32),
        scratch_types=[
            pltpu.VMEM((L,), jnp.int32),
            pltpu.VMEM((TR, M), jnp.float32),
        ],
    )(nrow16)


def _tc_body(nrow_ref, s_ref, o0_ref, o_ref):
    del o0_ref
    j = pl.program_id(1)
    nrow = nrow_ref[pl.program_id(0)]
    start = j * BR

    @pl.when((nrow == 0) & (j == 0))
    def _zero():
        o_ref[...] = jnp.zeros_like(o_ref)

    @pl.when(start < nrow)
    def _compute():
        x = s_ref[0]
        ss = jnp.sum(x * x, axis=-1, keepdims=True)
        r = 1.0 / jnp.maximum(jnp.sqrt(ss), 1e-12)
        e = jnp.exp(x * r)
        se = jnp.sum(e, axis=-1, keepdims=True)
        out = e / se

        @pl.when(start + BR > nrow)
        def _mask():
            rows = jax.lax.broadcasted_iota(jnp.int32, (BR, M), 0) + start
            o_ref[0] = jnp.where(rows < nrow, out, 0.0)

        @pl.when(start + BR <= nrow)
        def _full():
            o_ref[0] = out


def _clamped_index(b, j, nrow_ref):
    nrow = nrow_ref[b]
    last_active = jnp.maximum((nrow + BR - 1) // BR - 1, 0)
    return b, jnp.minimum(j, last_active), 0


def kernel(s, nrow_gt):
    nrow = nrow_gt.astype(jnp.int32)
    nrow16 = jnp.zeros((L,), jnp.int32).at[:B].set(nrow)
    out0 = _sc_zero_fill(nrow16).reshape(B, N, M)

    grid_spec = pltpu.PrefetchScalarGridSpec(
        num_scalar_prefetch=1,
        grid=(B, N // BR),
        in_specs=[
            pl.BlockSpec((1, BR, M), _clamped_index),
            pl.BlockSpec(memory_space=pltpu.MemorySpace.HBM),
        ],
        out_specs=pl.BlockSpec((1, BR, M), _clamped_index),
    )
    return pl.pallas_call(
        _tc_body,
        grid_spec=grid_spec,
        out_shape=jax.ShapeDtypeStruct((B, N, M), jnp.float32),
        input_output_aliases={2: 0},
    )(nrow, s, out0)


# speedup vs baseline: 1.3286x; 1.3286x over previous
import jax
import jax.numpy as jnp
from jax import lax
from jax.experimental import pallas as pl
from jax.experimental.pallas import tpu as pltpu
from jax.experimental.pallas import tpu_sc as plsc

B, N, M = 8, 2048, 2048
BR = 512
L = 16
TR = 16
R = B * N
NT = R // TR
NW = 32
TPW = NT // NW
TILES_PER_BATCH = N // TR
VPR = M // L


def _sc_zero_body(nrow_hbm, out_hbm, nrow_v, zbuf):
    wid = lax.axis_index("s") * 2 + lax.axis_index("c")

    pltpu.sync_copy(nrow_hbm, nrow_v)
    nrowf = nrow_v[...].astype(jnp.float32)
    lanes = jnp.arange(L, dtype=jnp.int32)

    z = jnp.zeros((L,), jnp.float32)

    def zrow(r, c):
        for k in range(VPR):
            zbuf[r, pl.ds(k * L, L)] = z
        return c

    lax.fori_loop(0, TR, zrow, jnp.int32(0))

    def tile_step(i, c):
        t = wid + NW * i
        b = t // TILES_PER_BATCH
        start = (t - b * TILES_PER_BATCH) * TR
        nrow_b = jnp.sum(jnp.where(lanes == b, nrowf, 0.0)).astype(jnp.int32)
        covered = BR * jnp.maximum((nrow_b + BR - 1) // BR, 1)

        @pl.when(start >= covered)
        def _():
            pltpu.sync_copy(zbuf, out_hbm.at[pl.ds(t * TR, TR)])

        return c

    lax.fori_loop(0, TPW, tile_step, jnp.int32(0))


def _sc_zero_fill(nrow16):
    mesh = plsc.VectorSubcoreMesh(core_axis_name="c", subcore_axis_name="s")
    return pl.kernel(
        _sc_zero_body,
        mesh=mesh,
        compiler_params=pltpu.CompilerParams(needs_layout_passes=False),
        out_type=jax.ShapeDtypeStruct((R, M), jnp.float32),
        scratch_types=[
            pltpu.VMEM((L,), jnp.int32),
            pltpu.VMEM((TR, M), jnp.float32),
        ],
    )(nrow16)


def _tc_body(nrow_ref, s_ref, o0_ref, o_ref):
    del o0_ref
    j = pl.program_id(1)
    nrow = nrow_ref[pl.program_id(0)]
    start = j * BR

    @pl.when((nrow == 0) & (j == 0))
    def _zero():
        o_ref[...] = jnp.zeros_like(o_ref)

    @pl.when(start < nrow)
    def _compute():
        x = s_ref[0]
        ss = jnp.sum(x * x, axis=-1, keepdims=True)
        r = 1.0 / jnp.maximum(jnp.sqrt(ss), 1e-12)
        e = jnp.exp(x * r)
        se = jnp.sum(e, axis=-1, keepdims=True)
        out = e / se

        @pl.when(start + BR > nrow)
        def _mask():
            rows = jax.lax.broadcasted_iota(jnp.int32, (BR, M), 0) + start
            o_ref[0] = jnp.where(rows < nrow, out, 0.0)

        @pl.when(start + BR <= nrow)
        def _full():
            o_ref[0] = out


def _clamped_index(b, j, nrow_ref):
    nrow = nrow_ref[b]
    last_active = jnp.maximum((nrow + BR - 1) // BR - 1, 0)
    return b, jnp.minimum(j, last_active), 0


def kernel(s, nrow_gt):
    nrow = nrow_gt.astype(jnp.int32)
    nrow16 = jnp.zeros((L,), jnp.int32).at[:B].set(nrow)
    out0 = _sc_zero_fill(nrow16).reshape(B, N, M)

    grid_spec = pltpu.PrefetchScalarGridSpec(
        num_scalar_prefetch=1,
        grid=(B, N // BR),
        in_specs=[
            pl.BlockSpec((1, BR, M), _clamped_index),
            pl.BlockSpec(memory_space=pltpu.MemorySpace.HBM),
        ],
        out_specs=pl.BlockSpec((1, BR, M), _clamped_index),
    )
    return pl.pallas_call(
        _tc_body,
        grid_spec=grid_spec,
        out_shape=jax.ShapeDtypeStruct((B, N, M), jnp.float32),
        input_output_aliases={2: 0},
    )(nrow, s, out0)
